# Initial kernel scaffold; baseline (speedup 1.0000x reference)
import jax
import jax.numpy as jnp
from jax.experimental import pallas as pl


def kernel(x, conv1_w, conv1_b, bn1_g, bn1_b, conv2_w, conv2_b, fc1_w, fc1_b, bn2_g, bn2_b, fc2_w, fc2_b, out_w, out_b):
    raise NotImplementedError("write your pallas kernel here")


# speedup vs baseline: 1.2471x; 1.2471x over previous
import numpy as np

import jax
import jax.numpy as jnp
from jax.experimental import pallas as pl
from jax.experimental.pallas import tpu as pltpu

BTILE = 128
EPS = 1e-5
BF = jnp.bfloat16


def _banded(w, win):
    cout, cin, k, _ = w.shape
    wout = win - k + 1
    sel = np.zeros((k, win, wout), np.float32)
    for kj in range(k):
        sel[kj, kj:kj + wout] = np.eye(wout, dtype=np.float32)
    r = jnp.einsum("ocik,kab->iacbo", w, jnp.asarray(sel))
    return r.reshape(k, win * cin, wout * cout)


def _pool_cols_packed(win, c, ncols, off):
    wout = win // 2
    m = np.zeros((win * c, ncols), np.float32)
    for j in range(wout):
        for ci in range(c):
            m[(2 * j) * c + ci, j * c + ci] = 1.0
            m[(2 * j + 1) * c + ci, off + j * c + ci] = 1.0
    return m


def _fs(shape):
    return pl.BlockSpec(shape, lambda i, _s=shape: (0,) * len(_s))


def _stage1(xb_ref, w_ref, b_ref, tp_ref, p_ref, s_ref, q_ref):
    z = jnp.dot(xb_ref[...], w_ref[...], preferred_element_type=jnp.float32)
    y = jnp.maximum(jnp.maximum(z[:, 0:144], z[:, 256:400]) + b_ref[...], 0.0)
    w = jnp.dot(y, tp_ref[...], preferred_element_type=jnp.float32)
    p = jnp.maximum(w[:, 0:128], w[:, 128:256])[:, 0:72]
    p_ref[...] = p.astype(BF)
    s_ref[0] = jnp.sum(p, axis=0, keepdims=True)
    q_ref[0] = jnp.sum(p * p, axis=0, keepdims=True)


def _stage2(p_ref, bn_ref, w2_ref, b2_ref, tp_ref, wf_ref, bf_ref,
            f_ref, s_ref, q_ref):
    pb = (p_ref[...].astype(jnp.float32) * bn_ref[0:1, :]
          + bn_ref[1:2, :]).astype(BF)
    z = jnp.dot(pb, w2_ref[...], preferred_element_type=jnp.float32)
    y = jnp.maximum(jnp.maximum(z[:, 0:384], z[:, 384:768]) + b2_ref[...], 0.0)
    w = jnp.dot(y, tp_ref[...], preferred_element_type=jnp.float32)
    p2 = jnp.maximum(w[:, 0:256], w[:, 256:512])
    f = jnp.maximum(
        jnp.dot(p2, wf_ref[...], preferred_element_type=jnp.float32)
        + bf_ref[...], 0.0)
    f_ref[...] = f.astype(BF)
    s_ref[0] = jnp.sum(f, axis=0, keepdims=True)
    q_ref[0] = jnp.sum(f * f, axis=0, keepdims=True)


def _stage3(f_ref, bn_ref, w2_ref, b2_ref, w3_ref, b3_ref, o_ref):
    h = f_ref[...].astype(jnp.float32) * bn_ref[0:1, :] + bn_ref[1:2, :]
    h = jnp.maximum(
        jnp.dot(h, w2_ref[...], preferred_element_type=jnp.float32)
        + b2_ref[...], 0.0)
    o_ref[...] = (jnp.dot(h, w3_ref[...],
                          preferred_element_type=jnp.float32) + b3_ref[...])


def kernel(x, conv1_w, conv1_b, bn1_g, bn1_b, conv2_w, conv2_b,
           fc1_w, fc1_b, bn2_g, bn2_b, fc2_w, fc2_b, out_w, out_b):
    n = x.shape[0]
    assert x.shape[1:] == (1, 28, 28) and n % BTILE == 0
    g = n // BTILE
    xs = x[:, 0, :, :]

    band1 = _banded(conv1_w, 28).reshape(140, 144)
    w1p = jnp.zeros((168, 400), jnp.float32)
    w1p = w1p.at[0:140, 0:144].set(band1)
    w1p = w1p.at[28:168, 256:400].set(band1)
    w1p = w1p.astype(BF)
    b1t = jnp.tile(conv1_b, 24).reshape(1, 144)
    tp1 = jnp.asarray(_pool_cols_packed(24, 6, 256, 128))

    band2 = _banded(conv2_w, 12)
    sel2 = np.zeros((5, 12, 8), np.float32)
    for k in range(5):
        for r in range(8):
            sel2[k, r + k, r] = 1.0
    t = jnp.einsum("kio,kpr->piro", band2, jnp.asarray(sel2))
    perm = np.array([0, 2, 4, 6, 1, 3, 5, 7])
    w2d = t[:, :, perm, :].reshape(864, 768).astype(BF)
    b2t = jnp.tile(conv2_b, 32).reshape(1, 384)
    te8 = _pool_cols_packed(8, 12, 96, 48)
    tp2 = np.zeros((384, 512), np.float32)
    for h2 in range(4):
        tp2[h2 * 96:(h2 + 1) * 96, h2 * 48:(h2 + 1) * 48] = te8[:, 0:48]
        tp2[h2 * 96:(h2 + 1) * 96, 256 + h2 * 48:256 + (h2 + 1) * 48] = \
            te8[:, 48:96]
    tp2 = jnp.asarray(tp2, jnp.float32)

    w1c = jnp.transpose(fc1_w.reshape(120, 12, 4, 4),
                        (2, 3, 1, 0)).reshape(192, 120)
    wfc = jnp.zeros((256, 120), jnp.float32).at[0:192, :].set(w1c)
    bf1 = fc1_b.reshape(1, 120)
    w2 = fc2_w.T
    bf2 = fc2_b.reshape(1, 60)
    w3 = out_w.T
    bo = out_b.reshape(1, 10)

    xb = jnp.concatenate([xs[:, d:d + 24:2, :] for d in range(6)], axis=-1)
    xb = xb.reshape(n * 12, 168).astype(BF)

    par = pltpu.CompilerParams(dimension_semantics=("parallel",))

    p1, s1, q1 = pl.pallas_call(
        _stage1,
        grid=(g,),
        in_specs=[pl.BlockSpec((BTILE * 12, 168), lambda i: (i, 0)),
                  _fs((168, 400)), _fs((1, 144)), _fs((144, 256))],
        out_specs=(pl.BlockSpec((BTILE * 12, 72), lambda i: (i, 0)),
                   pl.BlockSpec((1, 1, 72), lambda i: (i, 0, 0)),
                   pl.BlockSpec((1, 1, 72), lambda i: (i, 0, 0))),
        out_shape=(jax.ShapeDtypeStruct((n * 12, 72), BF),
                   jax.ShapeDtypeStruct((g, 1, 72), jnp.float32),
                   jax.ShapeDtypeStruct((g, 1, 72), jnp.float32)),
        compiler_params=par,
        cost_estimate=pl.CostEstimate(
            flops=int(n * 12 * 2 * (168 * 400 + 144 * 256)),
            transcendentals=0,
            bytes_accessed=int(n * 12 * (168 + 72) * 2 + 300_000)),
    )(xb, w1p, b1t, tp1)

    cnt1 = jnp.float32(n * 144)
    s6 = jnp.sum(s1, axis=(0, 1)).reshape(12, 6).sum(axis=0)
    q6 = jnp.sum(q1, axis=(0, 1)).reshape(12, 6).sum(axis=0)
    mean1 = s6 / cnt1
    var1 = q6 / cnt1 - mean1 * mean1
    g1 = bn1_g * jax.lax.rsqrt(var1 + EPS)
    bn1 = jnp.stack([jnp.tile(g1, 144), jnp.tile(bn1_b - mean1 * g1, 144)])

    p1f = p1.reshape(n, 864)

    f1, s2, q2 = pl.pallas_call(
        _stage2,
        grid=(g,),
        in_specs=[pl.BlockSpec((BTILE, 864), lambda i: (i, 0)),
                  _fs((2, 864)), _fs((864, 768)), _fs((1, 384)),
                  _fs((384, 512)), _fs((256, 120)), _fs((1, 120))],
        out_specs=(pl.BlockSpec((BTILE, 120), lambda i: (i, 0)),
                   pl.BlockSpec((1, 1, 120), lambda i: (i, 0, 0)),
                   pl.BlockSpec((1, 1, 120), lambda i: (i, 0, 0))),
        out_shape=(jax.ShapeDtypeStruct((n, 120), BF),
                   jax.ShapeDtypeStruct((g, 1, 120), jnp.float32),
                   jax.ShapeDtypeStruct((g, 1, 120), jnp.float32)),
        compiler_params=par,
        cost_estimate=pl.CostEstimate(
            flops=int(n * 2 * (864 * 768 + 384 * 512 + 256 * 120)),
            transcendentals=0,
            bytes_accessed=int(n * (864 + 120) * 2 + 2_000_000)),
    )(p1f, bn1, w2d, b2t, tp2, wfc, bf1)

    s120 = jnp.sum(s2, axis=(0, 1))
    q120 = jnp.sum(q2, axis=(0, 1))
    mean2 = s120 / jnp.float32(n)
    var2 = q120 / jnp.float32(n) - mean2 * mean2
    g2 = bn2_g * jax.lax.rsqrt(var2 + EPS)
    bn2 = jnp.stack([g2, bn2_b - mean2 * g2])

    gh = 8
    logits = pl.pallas_call(
        _stage3,
        grid=(gh,),
        in_specs=[pl.BlockSpec((n // gh, 120), lambda i: (i, 0)),
                  _fs((2, 120)), _fs((120, 60)), _fs((1, 60)),
                  _fs((60, 10)), _fs((1, 10))],
        out_specs=pl.BlockSpec((n // gh, 10), lambda i: (i, 0)),
        out_shape=jax.ShapeDtypeStruct((n, 10), jnp.float32),
        compiler_params=par,
        cost_estimate=pl.CostEstimate(flops=int(n * 2 * (120 * 60 + 60 * 10)),
                                      transcendentals=0,
                                      bytes_accessed=int(n * 300 + 20_000)),
    )(f1, bn2, w2, bf2, w3, bo)
    return logits


# speedup vs baseline: 6.5566x; 5.2577x over previous
import numpy as np

import jax
import jax.numpy as jnp
from jax.experimental import pallas as pl
from jax.experimental.pallas import tpu as pltpu

LB = 1024
EPS = 1e-5
BF = jnp.bfloat16


def _banded(w, win):
    cout, cin, k, _ = w.shape
    wout = win - k + 1
    sel = np.zeros((k, win, wout), np.float32)
    for kj in range(k):
        sel[kj, kj:kj + wout] = np.eye(wout, dtype=np.float32)
    r = jnp.einsum("ocik,kab->iacbo", w, jnp.asarray(sel))
    return r.reshape(k, win * cin, wout * cout)


def _pool_cols_packed(win, c, ncols, off):
    wout = win // 2
    m = np.zeros((win * c, ncols), np.float32)
    for j in range(wout):
        for ci in range(c):
            m[(2 * j) * c + ci, j * c + ci] = 1.0
            m[(2 * j + 1) * c + ci, off + j * c + ci] = 1.0
    return m


def _fs(shape):
    return pl.BlockSpec(shape, lambda i, _s=shape: (0,) * len(_s))


def _stage1(x_ref, w_ref, b_ref, tp_ref, p_ref, sq_ref):
    xw = jnp.concatenate([x_ref[:, j, :] for j in range(8)],
                         axis=-1).astype(BF)
    ps = []
    s_acc = None
    q_acc = None
    for h in range(12):
        win = xw[56 * h:56 * h + 168, :]
        z = jnp.dot(w_ref[...], win, preferred_element_type=jnp.float32)
        y = jnp.maximum(jnp.maximum(z[0:144, :], z[256:400, :])
                        + b_ref[...], 0.0)
        w = jnp.dot(tp_ref[...], y.astype(BF),
                    preferred_element_type=jnp.float32)
        ph = jnp.maximum(w[0:128, :], w[128:256, :])[0:72, :]
        ps.append(ph.astype(BF))
        s_acc = ph if s_acc is None else s_acc + ph
        q_acc = ph * ph if q_acc is None else q_acc + ph * ph
    p_ref[...] = jnp.concatenate(ps, axis=0)
    sq_ref[0] = jnp.concatenate(
        [jnp.sum(s_acc, axis=1, keepdims=True),
         jnp.sum(q_acc, axis=1, keepdims=True)], axis=1)


def _stage2(p_ref, w2_ref, bn_ref, b2_ref, tp_ref, wf_ref, bf_ref,
            f_ref, sq_ref):
    z = jnp.dot(w2_ref[...], p_ref[...],
                preferred_element_type=jnp.float32) + bn_ref[...]
    y = jnp.maximum(jnp.maximum(z[0:384, :], z[384:768, :])
                    + b2_ref[...], 0.0)
    w = jnp.dot(tp_ref[...], y.astype(BF),
                preferred_element_type=jnp.float32)
    p2 = jnp.maximum(w[0:256, :], w[256:512, :])
    f = jnp.maximum(
        jnp.dot(wf_ref[...], p2.astype(BF),
                preferred_element_type=jnp.float32) + bf_ref[...], 0.0)
    f_ref[...] = f.astype(BF)
    sq_ref[0] = jnp.concatenate(
        [jnp.sum(f, axis=1, keepdims=True),
         jnp.sum(f * f, axis=1, keepdims=True)], axis=1)


def _stage3(f_ref, w2_ref, b2_ref, w3_ref, b3_ref, o_ref):
    h = jnp.maximum(
        jnp.dot(w2_ref[...], f_ref[...],
                preferred_element_type=jnp.float32) + b2_ref[...], 0.0)
    o_ref[...] = (jnp.dot(w3_ref[...], h.astype(BF),
                          preferred_element_type=jnp.float32) + b3_ref[...])


def kernel(x, conv1_w, conv1_b, bn1_g, bn1_b, conv2_w, conv2_b,
           fc1_w, fc1_b, bn2_g, bn2_b, fc2_w, fc2_b, out_w, out_b):
    n = x.shape[0]
    assert x.shape[1:] == (1, 28, 28) and n % LB == 0
    g = n // LB

    xt = jnp.transpose(x, (1, 2, 3, 0)).reshape(784, n // 128, 128)

    band1 = _banded(conv1_w, 28).reshape(140, 144)
    w1p = jnp.zeros((168, 400), jnp.float32)
    w1p = w1p.at[0:140, 0:144].set(band1)
    w1p = w1p.at[28:168, 256:400].set(band1)
    w1t = w1p.T.astype(BF)
    b1c = jnp.tile(conv1_b, 24).reshape(144, 1)
    tp1t = jnp.asarray(_pool_cols_packed(24, 6, 256, 128).T, BF)

    band2 = _banded(conv2_w, 12)
    sel2 = np.zeros((5, 12, 8), np.float32)
    for k in range(5):
        for r in range(8):
            sel2[k, r + k, r] = 1.0
    t = jnp.einsum("kio,kpr->piro", band2, jnp.asarray(sel2))
    perm = np.array([0, 2, 4, 6, 1, 3, 5, 7])
    w2d = t[:, :, perm, :].reshape(864, 768)
    b2c = jnp.tile(conv2_b, 32).reshape(384, 1)
    te8 = _pool_cols_packed(8, 12, 96, 48)
    tp2 = np.zeros((384, 512), np.float32)
    for h2 in range(4):
        tp2[h2 * 96:(h2 + 1) * 96, h2 * 48:(h2 + 1) * 48] = te8[:, 0:48]
        tp2[h2 * 96:(h2 + 1) * 96, 256 + h2 * 48:256 + (h2 + 1) * 48] = \
            te8[:, 48:96]
    tp2t = jnp.asarray(tp2.T, BF)

    w1c = jnp.transpose(fc1_w.reshape(120, 12, 4, 4),
                        (2, 3, 1, 0)).reshape(192, 120)
    wfct = jnp.zeros((256, 120), jnp.float32).at[0:192, :].set(w1c).T
    bf1c = fc1_b.reshape(120, 1)

    par = pltpu.CompilerParams(dimension_semantics=("parallel",))

    p1t, sq1 = pl.pallas_call(
        _stage1,
        grid=(g,),
        in_specs=[pl.BlockSpec((784, LB // 128, 128), lambda i: (0, i, 0)),
                  _fs((400, 168)), _fs((144, 1)), _fs((256, 144))],
        out_specs=(pl.BlockSpec((864, LB), lambda i: (0, i)),
                   pl.BlockSpec((1, 72, 2), lambda i: (i, 0, 0))),
        out_shape=(jax.ShapeDtypeStruct((864, n), BF),
                   jax.ShapeDtypeStruct((g, 72, 2), jnp.float32)),
        compiler_params=par,
        cost_estimate=pl.CostEstimate(
            flops=int(n * 12 * 2 * (168 * 400 + 144 * 256)),
            transcendentals=0,
            bytes_accessed=int(n * (784 + 864) * 2 + 300_000)),
    )(xt, w1t, b1c, tp1t)

    cnt1 = jnp.float32(n * 144)
    sq = jnp.sum(sq1, axis=0).reshape(12, 6, 2).sum(axis=0)
    mean1 = sq[:, 0] / cnt1
    var1 = sq[:, 1] / cnt1 - mean1 * mean1
    g1 = bn1_g * jax.lax.rsqrt(var1 + EPS)
    w2g = (w2d.T * jnp.tile(g1, 144)[None, :]).astype(BF)
    bshift = jnp.tile(bn1_b - mean1 * g1, 144).reshape(864, 1)
    bn1c = jnp.dot(w2d.T, bshift)

    f1t, sq2 = pl.pallas_call(
        _stage2,
        grid=(g,),
        in_specs=[pl.BlockSpec((864, LB), lambda i: (0, i)),
                  _fs((768, 864)), _fs((768, 1)), _fs((384, 1)),
                  _fs((512, 384)), _fs((120, 256)), _fs((120, 1))],
        out_specs=(pl.BlockSpec((120, LB), lambda i: (0, i)),
                   pl.BlockSpec((1, 120, 2), lambda i: (i, 0, 0))),
        out_shape=(jax.ShapeDtypeStruct((120, n), BF),
                   jax.ShapeDtypeStruct((g, 120, 2), jnp.float32)),
        compiler_params=par,
        cost_estimate=pl.CostEstimate(
            flops=int(n * 2 * (864 * 768 + 384 * 512 + 256 * 120)),
            transcendentals=0,
            bytes_accessed=int(n * (864 + 120) * 2 + 2_000_000)),
    )(p1t, w2g, bn1c, b2c, tp2t, wfct.astype(BF), bf1c)

    sqb = jnp.sum(sq2, axis=0)
    mean2 = sqb[:, 0] / jnp.float32(n)
    var2 = sqb[:, 1] / jnp.float32(n) - mean2 * mean2
    g2 = bn2_g * jax.lax.rsqrt(var2 + EPS)
    w2h = (fc2_w * g2[None, :]).astype(BF)
    bh = (jnp.dot(fc2_w, (bn2_b - mean2 * g2).reshape(120, 1))
          + fc2_b.reshape(60, 1))

    logits_t = pl.pallas_call(
        _stage3,
        grid=(g,),
        in_specs=[pl.BlockSpec((120, LB), lambda i: (0, i)),
                  _fs((60, 120)), _fs((60, 1)), _fs((10, 60)), _fs((10, 1))],
        out_specs=pl.BlockSpec((10, LB), lambda i: (0, i)),
        out_shape=jax.ShapeDtypeStruct((10, n), jnp.float32),
        compiler_params=par,
        cost_estimate=pl.CostEstimate(flops=int(n * 2 * (120 * 60 + 60 * 10)),
                                      transcendentals=0,
                                      bytes_accessed=int(n * 300 + 20_000)),
    )(f1t, w2h, bh, out_w.astype(BF), out_b.reshape(10, 1))
    return logits_t.T
